# baseline (speedup 1.0000x reference)
import functools

import jax
import jax.numpy as jnp
from jax import lax
from jax.experimental import pallas as pl
from jax.experimental.pallas import tpu as pltpu
from jax.experimental.pallas import tpu_sc as plsc

N_NODES = 10000
N_EDGES = 320000
HP = 8
EP = 8
NW = 32
CH = 1280
NCH = N_EDGES // CH
NPAIR = 4
BET = 64000
NBT = N_EDGES // BET


@functools.lru_cache(maxsize=None)
def _sc_mesh():
    return plsc.VectorSubcoreMesh(
        core_axis_name="c", subcore_axis_name="s", num_cores=2,
        num_subcores=16)


_SC_PARAMS = pltpu.CompilerParams(needs_layout_passes=False)


def _mm(a, b):
    return lax.dot_general(
        a, b, (((1,), (0,)), ((), ())),
        precision=lax.Precision.DEFAULT, preferred_element_type=jnp.float32)


def _gather_body(h_hbm, src_hbm, dst_hbm, g_hbm, tab, idx_v,
                 gbufA, gbufB, sem_i, sem_o):
    c = lax.axis_index("c")
    s = lax.axis_index("s")
    wid = s * 2 + c
    nk = (NCH + NW - 1 - wid) // NW

    def fire_idx(k, b):
        base = (wid + NW * k) * CH
        pltpu.async_copy(dst_hbm.at[pl.ds(base, CH)],
                         idx_v.at[pl.ds((b * 2) * CH, CH)], sem_i)
        pltpu.async_copy(src_hbm.at[pl.ds(base, CH)],
                         idx_v.at[pl.ds((b * 2 + 1) * CH, CH)], sem_i)

    fire_idx(0, 0)
    pltpu.sync_copy(h_hbm.at[pl.ds(0, 5 * N_NODES)], tab)

    def do_chunk(k, gbuf, b):
        pltpu.make_async_copy(
            dst_hbm.at[pl.ds(0, CH)], idx_v.at[pl.ds(0, CH)], sem_i).wait()
        pltpu.make_async_copy(
            dst_hbm.at[pl.ds(0, CH)], idx_v.at[pl.ds(0, CH)], sem_i).wait()

        @pl.when(k + 1 < nk)
        def _prefetch():
            fire_idx(k + 1, 1 - b)

        @pl.when(k >= 2)
        def _drain():
            pltpu.make_async_copy(
                gbufA, g_hbm.at[:, pl.ds(0, CH)], sem_o).wait()

        ib = b * 2 * CH

        @plsc.parallel_loop(0, CH // 16, unroll=8)
        def vec(i):
            off = i * 16
            di = idx_v[pl.ds(ib + off, 16)]
            si = idx_v[pl.ds(ib + CH + off, 16)]
            for col in range(5):
                gbuf[col, pl.ds(off, 16)] = plsc.load_gather(
                    tab, [di + col * N_NODES])
                gbuf[col + 5, pl.ds(off, 16)] = plsc.load_gather(
                    tab, [si + col * N_NODES])

        base = (wid + NW * k) * CH
        pltpu.async_copy(gbuf, g_hbm.at[:, pl.ds(base, CH)], sem_o)

    def pair(k2, carry):
        k = k2 * 2

        @pl.when(k < nk)
        def _a():
            do_chunk(k, gbufA, 0)

        @pl.when(k + 1 < nk)
        def _b():
            do_chunk(k + 1, gbufB, 1)

        return carry

    lax.fori_loop(0, NPAIR, pair, 0)
    pltpu.make_async_copy(gbufA, g_hbm.at[:, pl.ds(0, CH)], sem_o).wait()
    pltpu.make_async_copy(gbufA, g_hbm.at[:, pl.ds(0, CH)], sem_o).wait()


@functools.lru_cache(maxsize=None)
def _gather_kernel():
    return pl.kernel(
        _gather_body,
        out_type=jax.ShapeDtypeStruct((10, N_EDGES), jnp.float32),
        mesh=_sc_mesh(),
        compiler_params=_SC_PARAMS,
        scratch_types=[
            pltpu.VMEM((5 * N_NODES,), jnp.float32),
            pltpu.VMEM((4 * CH,), jnp.int32),
            pltpu.VMEM((10, CH), jnp.float32),
            pltpu.VMEM((10, CH), jnp.float32),
            pltpu.SemaphoreType.DMA,
            pltpu.SemaphoreType.DMA,
        ],
    )


def _gather(h_t, src, dst):
    return _gather_kernel()(h_t.reshape(-1), src, dst)


def _scatter_body(et_hbm, dst_hbm, out_hbm, acc, idx_v,
                  ebufA, ebufB, sem_i):
    c = lax.axis_index("c")
    s = lax.axis_index("s")
    wid = s * 2 + c
    nk = (NCH + NW - 1 - wid) // NW

    def fire(k, b, ebuf):
        base = (wid + NW * k) * CH
        pltpu.async_copy(dst_hbm.at[pl.ds(base, CH)],
                         idx_v.at[pl.ds(b * CH, CH)], sem_i)
        pltpu.async_copy(et_hbm.at[:, pl.ds(base, CH)], ebuf, sem_i)

    fire(0, 0, ebufA)
    zero16 = jnp.zeros((16,), jnp.float32)

    @plsc.parallel_loop(0, N_NODES // 16, unroll=4)
    def zacc(i):
        off = i * 16
        for r in range(4):
            acc[r, pl.ds(off, 16)] = zero16

    def do_chunk(k, ebuf, other, b):
        pltpu.make_async_copy(
            dst_hbm.at[pl.ds(0, CH)], idx_v.at[pl.ds(0, CH)], sem_i).wait()
        pltpu.make_async_copy(
            et_hbm.at[:, pl.ds(0, CH)], ebufA, sem_i).wait()

        @pl.when(k + 1 < nk)
        def _prefetch():
            fire(k + 1, 1 - b, other)

        ib = b * CH

        @plsc.parallel_loop(0, CH // 16, unroll=8)
        def vec(i):
            off = i * 16
            di = idx_v[pl.ds(ib + off, 16)]
            for col in range(4):
                cv = jnp.full((16,), col, jnp.int32)
                vals = ebuf[col, pl.ds(off, 16)]
                plsc.addupdate_scatter(acc, [cv, di], vals)

    def pair(k2, carry):
        k = k2 * 2

        @pl.when(k < nk)
        def _a():
            do_chunk(k, ebufA, ebufB, 0)

        @pl.when(k + 1 < nk)
        def _b():
            do_chunk(k + 1, ebufB, ebufA, 1)

        return carry

    lax.fori_loop(0, NPAIR, pair, 0)
    pltpu.sync_copy(acc, out_hbm.at[wid])


@functools.lru_cache(maxsize=None)
def _scatter_kernel():
    return pl.kernel(
        _scatter_body,
        out_type=jax.ShapeDtypeStruct((NW, 4, N_NODES), jnp.float32),
        mesh=_sc_mesh(),
        compiler_params=_SC_PARAMS,
        scratch_types=[
            pltpu.VMEM((4, N_NODES), jnp.float32),
            pltpu.VMEM((2 * CH,), jnp.int32),
            pltpu.VMEM((4, CH), jnp.float32),
            pltpu.VMEM((4, CH), jnp.float32),
            pltpu.SemaphoreType.DMA,
        ],
    )


def _scatter(et2, dst):
    return _scatter_kernel()(et2, dst)


def _full(shape):
    return pl.BlockSpec(shape, lambda i: tuple(0 for _ in shape))


def _node_enc_kernel(xt_ref, tw0_ref, tw1_ref, h_ref):
    m = jnp.maximum(_mm(tw0_ref[...], xt_ref[...]), 0.0)
    h_ref[...] = jnp.maximum(_mm(tw1_ref[...], m), 0.0)


def _node_enc(x_t, tw0, tw1p):
    return pl.pallas_call(
        _node_enc_kernel,
        grid=(1,),
        in_specs=[_full(x_t.shape), _full(tw0.shape), _full(tw1p.shape)],
        out_specs=_full((HP, N_NODES)),
        out_shape=jax.ShapeDtypeStruct((HP, N_NODES), jnp.float32),
    )(x_t, tw0, tw1p)


def _edge_enc_kernel(ea_ref, tw0_ref, tw1_ref, e_ref):
    m = jnp.maximum(_mm(tw0_ref[...], ea_ref[...]), 0.0)
    e_ref[...] = jnp.maximum(_mm(tw1_ref[...], m), 0.0)


def _edge_enc(ea_t, tw0, tw1p):
    return pl.pallas_call(
        _edge_enc_kernel,
        grid=(NBT,),
        in_specs=[pl.BlockSpec((16, BET), lambda i: (0, i)),
                  _full(tw0.shape), _full(tw1p.shape)],
        out_specs=pl.BlockSpec((4, BET), lambda i: (0, i)),
        out_shape=jax.ShapeDtypeStruct((4, N_EDGES), jnp.float32),
    )(ea_t, tw0, tw1p)


def _rel_kernel(g_ref, e_ref, tw1, b1, tw2, b2, tw3, b3, et_ref, en_ref):
    ge = jnp.concatenate([g_ref[...], e_ref[...]], axis=0)
    m1 = jnp.maximum(_mm(tw1[...], ge) + b1[...], 0.0)
    m2 = jnp.maximum(_mm(tw2[...], m1) + b2[...], 0.0)
    et = _mm(tw3[...], m2) + b3[...]
    et_ref[...] = et
    en_ref[...] = 0.5 * e_ref[...] + 0.5 * jnp.maximum(et, 0.0)


def _rel(g2, e_t, tw1, b1, tw2, b2, tw3, b3):
    return pl.pallas_call(
        _rel_kernel,
        grid=(NBT,),
        in_specs=[pl.BlockSpec((10, BET), lambda i: (0, i)),
                  pl.BlockSpec((4, BET), lambda i: (0, i)),
                  _full(tw1.shape), _full(b1.shape),
                  _full(tw2.shape), _full(b2.shape),
                  _full(tw3.shape), _full(b3.shape)],
        out_specs=[pl.BlockSpec((4, BET), lambda i: (0, i)),
                   pl.BlockSpec((4, BET), lambda i: (0, i))],
        out_shape=[jax.ShapeDtypeStruct((4, N_EDGES), jnp.float32),
                   jax.ShapeDtypeStruct((4, N_EDGES), jnp.float32)],
    )(g2, e_t, tw1, b1, tw2, b2, tw3, b3)


def _obj_kernel(h_ref, p_ref, twh, twa, b1, tw2, b2, tw3, b3, hn_ref):
    aggr = jnp.sum(p_ref[...], axis=0)
    m1 = jnp.maximum(
        _mm(twh[...], h_ref[...]) + _mm(twa[...], aggr) + b1[...], 0.0)
    m2 = jnp.maximum(_mm(tw2[...], m1) + b2[...], 0.0)
    ht = _mm(tw3[...], m2) + b3[...]
    hn_ref[...] = 0.5 * h_ref[...] + 0.5 * jnp.maximum(ht, 0.0)


def _obj(h_t, p, twh, twa, b1, tw2, b2, tw3, b3):
    return pl.pallas_call(
        _obj_kernel,
        grid=(1,),
        in_specs=[_full((HP, N_NODES)), _full((NW, 4, N_NODES)),
                  _full(twh.shape), _full(twa.shape), _full(b1.shape),
                  _full(tw2.shape), _full(b2.shape),
                  _full(tw3.shape), _full(b3.shape)],
        out_specs=_full((HP, N_NODES)),
        out_shape=jax.ShapeDtypeStruct((HP, N_NODES), jnp.float32),
    )(h_t, p, twh, twa, b1, tw2, b2, tw3, b3)


def _rel_final_kernel(g_ref, e0, e1, e2, tw1, b1, tw2, b2, tw3, b3,
                      fw1, fb1, fw2, fb2, fw3, fb3, w_ref):
    ge = jnp.concatenate([g_ref[...], e2[...]], axis=0)
    m1 = jnp.maximum(_mm(tw1[...], ge) + b1[...], 0.0)
    m2 = jnp.maximum(_mm(tw2[...], m1) + b2[...], 0.0)
    et = _mm(tw3[...], m2) + b3[...]
    e3 = 0.5 * e2[...] + 0.5 * jnp.maximum(et, 0.0)
    ecat = jnp.concatenate([e0[...], e1[...], e2[...], e3], axis=0)
    f1 = jnp.maximum(_mm(fw1[...], ecat) + fb1[...], 0.0)
    f2 = jnp.maximum(_mm(fw2[...], f1) + fb2[...], 0.0)
    z = _mm(fw3[...], f2) + fb3[...]
    w_ref[...] = jax.nn.sigmoid(z)


def _rel_final(g2, es, relw, finw):
    espec = pl.BlockSpec((4, BET), lambda i: (0, i))
    specs = [pl.BlockSpec((10, BET), lambda i: (0, i)), espec, espec, espec]
    specs += [_full(w.shape) for w in (*relw, *finw)]
    return pl.pallas_call(
        _rel_final_kernel,
        grid=(NBT,),
        in_specs=specs,
        out_specs=pl.BlockSpec((EP, BET), lambda i: (0, i)),
        out_shape=jax.ShapeDtypeStruct((EP, N_EDGES), jnp.float32),
    )(g2, *es, *relw, *finw)


def _pad_cols(w, n):
    return jnp.pad(w, ((0, 0), (0, n - w.shape[1])))


def _pad_rows(w, n):
    return jnp.pad(w, ((0, n - w.shape[0]), (0, 0)))


def kernel(params, x, edge_index, edge_attr):
    src = edge_index[0]
    dst = edge_index[1]
    x_t = x.T
    ea_t = edge_attr.T

    ne0, _ = params['node_enc'][0]
    ne1, _ = params['node_enc'][1]
    ee0, _ = params['edge_enc'][0]
    ee1, _ = params['edge_enc'][1]
    tne0 = ne0.T
    tne1p = _pad_rows(ne1.T, HP)
    tee0 = ee0.T
    tee1p = ee1.T

    h_t = _node_enc(x_t, tne0, tne1p)
    e_t = _edge_enc(ea_t, tee0, tee1p)
    e_list = [e_t]

    def rel_weights(lp):
        (rw1, rb1), (rw2, rb2), (rw3, rb3) = lp['rel']
        return (rw1.T,
                rb1[:, None], rw2.T, rb2[:, None], rw3.T, rb3[:, None])

    for lp in params['resin'][:-1]:
        (ow1, ob1), (ow2, ob2), (ow3, ob3) = lp['obj']
        g2 = _gather(h_t, src, dst)
        et2, e_t = _rel(g2, e_t, *rel_weights(lp))
        part = _scatter(et2, dst)

        tow1 = ow1.T
        towh = _pad_cols(tow1[:, 0:5], HP)
        towa = tow1[:, 5:9]
        tow3 = _pad_rows(ow3.T, HP)
        tob3 = _pad_rows(ob3[:, None], HP)
        h_t = _obj(h_t, part, towh, towa, ob1[:, None], ow2.T,
                   ob2[:, None], tow3, tob3)
        e_list.append(e_t)

    (ww1, wb1), (ww2, wb2), (ww3, wb3) = params['W']
    finw = (ww1.T,
            wb1[:, None], ww2.T, wb2[:, None],
            _pad_rows(ww3.T, EP), _pad_rows(wb3[:, None], EP))

    g2 = _gather(h_t, src, dst)
    w8 = _rel_final(g2, e_list, rel_weights(params['resin'][-1]), finw)
    return w8[0][:, None]


# speedup vs baseline: 1.0295x; 1.0295x over previous
import functools

import jax
import jax.numpy as jnp
from jax import lax
from jax.experimental import pallas as pl
from jax.experimental.pallas import tpu as pltpu
from jax.experimental.pallas import tpu_sc as plsc

N_NODES = 10000
N_EDGES = 320000
HP = 8
EP = 8
NW = 32
CH = 1280
NCH = N_EDGES // CH
NPAIR = 4
BET = 64000
NBT = N_EDGES // BET


@functools.lru_cache(maxsize=None)
def _sc_mesh():
    return plsc.VectorSubcoreMesh(
        core_axis_name="c", subcore_axis_name="s", num_cores=2,
        num_subcores=16)


_SC_PARAMS = pltpu.CompilerParams(needs_layout_passes=False)


def _mm(a, b):
    return lax.dot_general(
        a, b, (((1,), (0,)), ((), ())),
        precision=lax.Precision.DEFAULT, preferred_element_type=jnp.float32)


def _gather_body(h_hbm, src_hbm, dst_hbm, g_hbm, tab, idx_v,
                 gbufA, gbufB, sem_i, sem_o):
    c = lax.axis_index("c")
    s = lax.axis_index("s")
    wid = s * 2 + c
    nk = (NCH + NW - 1 - wid) // NW

    def fire_idx(k, b):
        base = (wid + NW * k) * CH
        pltpu.async_copy(dst_hbm.at[pl.ds(base, CH)],
                         idx_v.at[pl.ds((b * 2) * CH, CH)], sem_i)
        pltpu.async_copy(src_hbm.at[pl.ds(base, CH)],
                         idx_v.at[pl.ds((b * 2 + 1) * CH, CH)], sem_i)

    fire_idx(0, 0)
    pltpu.sync_copy(h_hbm.at[pl.ds(0, 5 * N_NODES)], tab)

    def do_chunk(k, gbuf, b):
        pltpu.make_async_copy(
            dst_hbm.at[pl.ds(0, CH)], idx_v.at[pl.ds(0, CH)], sem_i).wait()
        pltpu.make_async_copy(
            dst_hbm.at[pl.ds(0, CH)], idx_v.at[pl.ds(0, CH)], sem_i).wait()

        @pl.when(k + 1 < nk)
        def _prefetch():
            fire_idx(k + 1, 1 - b)

        @pl.when(k >= 2)
        def _drain():
            pltpu.make_async_copy(
                gbufA, g_hbm.at[:, pl.ds(0, CH)], sem_o).wait()

        ib = b * 2 * CH

        @plsc.parallel_loop(0, CH // 16, unroll=4)
        def vec(i):
            off = i * 16
            di = idx_v[pl.ds(ib + off, 16)]
            si = idx_v[pl.ds(ib + CH + off, 16)]
            for col in range(5):
                gbuf[col, pl.ds(off, 16)] = plsc.load_gather(
                    tab, [di + col * N_NODES])
                gbuf[col + 5, pl.ds(off, 16)] = plsc.load_gather(
                    tab, [si + col * N_NODES])

        base = (wid + NW * k) * CH
        pltpu.async_copy(gbuf, g_hbm.at[:, pl.ds(base, CH)], sem_o)

    def pair(k2, carry):
        k = k2 * 2

        @pl.when(k < nk)
        def _a():
            do_chunk(k, gbufA, 0)

        @pl.when(k + 1 < nk)
        def _b():
            do_chunk(k + 1, gbufB, 1)

        return carry

    lax.fori_loop(0, NPAIR, pair, 0)
    pltpu.make_async_copy(gbufA, g_hbm.at[:, pl.ds(0, CH)], sem_o).wait()
    pltpu.make_async_copy(gbufA, g_hbm.at[:, pl.ds(0, CH)], sem_o).wait()


@functools.lru_cache(maxsize=None)
def _gather_kernel():
    return pl.kernel(
        _gather_body,
        out_type=jax.ShapeDtypeStruct((10, N_EDGES), jnp.float32),
        mesh=_sc_mesh(),
        compiler_params=_SC_PARAMS,
        scratch_types=[
            pltpu.VMEM((5 * N_NODES,), jnp.float32),
            pltpu.VMEM((4 * CH,), jnp.int32),
            pltpu.VMEM((10, CH), jnp.float32),
            pltpu.VMEM((10, CH), jnp.float32),
            pltpu.SemaphoreType.DMA,
            pltpu.SemaphoreType.DMA,
        ],
    )


def _gather(h_t, src, dst):
    return _gather_kernel()(h_t.reshape(-1), src, dst)


def _scatter_body(et_hbm, dst_hbm, out_hbm, acc, idx_v,
                  ebufA, ebufB, sem_i):
    c = lax.axis_index("c")
    s = lax.axis_index("s")
    wid = s * 2 + c
    nk = (NCH + NW - 1 - wid) // NW

    def fire(k, b, ebuf):
        base = (wid + NW * k) * CH
        pltpu.async_copy(dst_hbm.at[pl.ds(base, CH)],
                         idx_v.at[pl.ds(b * CH, CH)], sem_i)
        pltpu.async_copy(et_hbm.at[:, pl.ds(base, CH)], ebuf, sem_i)

    fire(0, 0, ebufA)
    zero16 = jnp.zeros((16,), jnp.float32)

    @plsc.parallel_loop(0, N_NODES // 16, unroll=4)
    def zacc(i):
        off = i * 16
        for r in range(4):
            acc[r, pl.ds(off, 16)] = zero16

    def do_chunk(k, ebuf, other, b):
        pltpu.make_async_copy(
            dst_hbm.at[pl.ds(0, CH)], idx_v.at[pl.ds(0, CH)], sem_i).wait()
        pltpu.make_async_copy(
            et_hbm.at[:, pl.ds(0, CH)], ebufA, sem_i).wait()

        @pl.when(k + 1 < nk)
        def _prefetch():
            fire(k + 1, 1 - b, other)

        ib = b * CH

        @plsc.parallel_loop(0, CH // 16, unroll=4)
        def vec(i):
            off = i * 16
            di = idx_v[pl.ds(ib + off, 16)]
            for col in range(4):
                cv = jnp.full((16,), col, jnp.int32)
                vals = ebuf[col, pl.ds(off, 16)]
                plsc.addupdate_scatter(acc, [cv, di], vals)

    def pair(k2, carry):
        k = k2 * 2

        @pl.when(k < nk)
        def _a():
            do_chunk(k, ebufA, ebufB, 0)

        @pl.when(k + 1 < nk)
        def _b():
            do_chunk(k + 1, ebufB, ebufA, 1)

        return carry

    lax.fori_loop(0, NPAIR, pair, 0)
    pltpu.sync_copy(acc, out_hbm.at[wid])


@functools.lru_cache(maxsize=None)
def _scatter_kernel():
    return pl.kernel(
        _scatter_body,
        out_type=jax.ShapeDtypeStruct((NW, 4, N_NODES), jnp.float32),
        mesh=_sc_mesh(),
        compiler_params=_SC_PARAMS,
        scratch_types=[
            pltpu.VMEM((4, N_NODES), jnp.float32),
            pltpu.VMEM((2 * CH,), jnp.int32),
            pltpu.VMEM((4, CH), jnp.float32),
            pltpu.VMEM((4, CH), jnp.float32),
            pltpu.SemaphoreType.DMA,
        ],
    )


def _scatter(et2, dst):
    return _scatter_kernel()(et2, dst)


def _full(shape):
    return pl.BlockSpec(shape, lambda i: tuple(0 for _ in shape))


def _node_enc_kernel(xt_ref, tw0_ref, tw1_ref, h_ref):
    m = jnp.maximum(_mm(tw0_ref[...], xt_ref[...]), 0.0)
    h_ref[...] = jnp.maximum(_mm(tw1_ref[...], m), 0.0)


def _node_enc(x_t, tw0, tw1p):
    return pl.pallas_call(
        _node_enc_kernel,
        grid=(1,),
        in_specs=[_full(x_t.shape), _full(tw0.shape), _full(tw1p.shape)],
        out_specs=_full((HP, N_NODES)),
        out_shape=jax.ShapeDtypeStruct((HP, N_NODES), jnp.float32),
    )(x_t, tw0, tw1p)


def _edge_enc_kernel(ea_ref, tw0_ref, tw1_ref, e_ref):
    m = jnp.maximum(_mm(tw0_ref[...], ea_ref[...]), 0.0)
    e_ref[...] = jnp.maximum(_mm(tw1_ref[...], m), 0.0)


def _edge_enc(ea_t, tw0, tw1p):
    return pl.pallas_call(
        _edge_enc_kernel,
        grid=(NBT,),
        in_specs=[pl.BlockSpec((16, BET), lambda i: (0, i)),
                  _full(tw0.shape), _full(tw1p.shape)],
        out_specs=pl.BlockSpec((4, BET), lambda i: (0, i)),
        out_shape=jax.ShapeDtypeStruct((4, N_EDGES), jnp.float32),
    )(ea_t, tw0, tw1p)


def _rel_kernel(g_ref, e_ref, tw1, b1, tw2, b2, tw3, b3, et_ref, en_ref):
    ge = jnp.concatenate([g_ref[...], e_ref[...]], axis=0)
    m1 = jnp.maximum(_mm(tw1[...], ge) + b1[...], 0.0)
    m2 = jnp.maximum(_mm(tw2[...], m1) + b2[...], 0.0)
    et = _mm(tw3[...], m2) + b3[...]
    et_ref[...] = et
    en_ref[...] = 0.5 * e_ref[...] + 0.5 * jnp.maximum(et, 0.0)


def _rel(g2, e_t, tw1, b1, tw2, b2, tw3, b3):
    return pl.pallas_call(
        _rel_kernel,
        grid=(NBT,),
        in_specs=[pl.BlockSpec((10, BET), lambda i: (0, i)),
                  pl.BlockSpec((4, BET), lambda i: (0, i)),
                  _full(tw1.shape), _full(b1.shape),
                  _full(tw2.shape), _full(b2.shape),
                  _full(tw3.shape), _full(b3.shape)],
        out_specs=[pl.BlockSpec((4, BET), lambda i: (0, i)),
                   pl.BlockSpec((4, BET), lambda i: (0, i))],
        out_shape=[jax.ShapeDtypeStruct((4, N_EDGES), jnp.float32),
                   jax.ShapeDtypeStruct((4, N_EDGES), jnp.float32)],
    )(g2, e_t, tw1, b1, tw2, b2, tw3, b3)


def _obj_kernel(h_ref, p_ref, twh, twa, b1, tw2, b2, tw3, b3, hn_ref):
    aggr = jnp.sum(p_ref[...], axis=0)
    m1 = jnp.maximum(
        _mm(twh[...], h_ref[...]) + _mm(twa[...], aggr) + b1[...], 0.0)
    m2 = jnp.maximum(_mm(tw2[...], m1) + b2[...], 0.0)
    ht = _mm(tw3[...], m2) + b3[...]
    hn_ref[...] = 0.5 * h_ref[...] + 0.5 * jnp.maximum(ht, 0.0)


def _obj(h_t, p, twh, twa, b1, tw2, b2, tw3, b3):
    return pl.pallas_call(
        _obj_kernel,
        grid=(1,),
        in_specs=[_full((HP, N_NODES)), _full((NW, 4, N_NODES)),
                  _full(twh.shape), _full(twa.shape), _full(b1.shape),
                  _full(tw2.shape), _full(b2.shape),
                  _full(tw3.shape), _full(b3.shape)],
        out_specs=_full((HP, N_NODES)),
        out_shape=jax.ShapeDtypeStruct((HP, N_NODES), jnp.float32),
    )(h_t, p, twh, twa, b1, tw2, b2, tw3, b3)


def _rel_final_kernel(g_ref, e0, e1, e2, tw1, b1, tw2, b2, tw3, b3,
                      fw1, fb1, fw2, fb2, fw3, fb3, w_ref):
    ge = jnp.concatenate([g_ref[...], e2[...]], axis=0)
    m1 = jnp.maximum(_mm(tw1[...], ge) + b1[...], 0.0)
    m2 = jnp.maximum(_mm(tw2[...], m1) + b2[...], 0.0)
    et = _mm(tw3[...], m2) + b3[...]
    e3 = 0.5 * e2[...] + 0.5 * jnp.maximum(et, 0.0)
    ecat = jnp.concatenate([e0[...], e1[...], e2[...], e3], axis=0)
    f1 = jnp.maximum(_mm(fw1[...], ecat) + fb1[...], 0.0)
    f2 = jnp.maximum(_mm(fw2[...], f1) + fb2[...], 0.0)
    z = _mm(fw3[...], f2) + fb3[...]
    w_ref[...] = jax.nn.sigmoid(z)


def _rel_final(g2, es, relw, finw):
    espec = pl.BlockSpec((4, BET), lambda i: (0, i))
    specs = [pl.BlockSpec((10, BET), lambda i: (0, i)), espec, espec, espec]
    specs += [_full(w.shape) for w in (*relw, *finw)]
    return pl.pallas_call(
        _rel_final_kernel,
        grid=(NBT,),
        in_specs=specs,
        out_specs=pl.BlockSpec((1, BET), lambda i: (0, i)),
        out_shape=jax.ShapeDtypeStruct((1, N_EDGES), jnp.float32),
    )(g2, *es, *relw, *finw)


def _pad_cols(w, n):
    return jnp.pad(w, ((0, 0), (0, n - w.shape[1])))


def _pad_rows(w, n):
    return jnp.pad(w, ((0, n - w.shape[0]), (0, 0)))


def kernel(params, x, edge_index, edge_attr):
    src = edge_index[0]
    dst = edge_index[1]
    x_t = x.T
    ea_t = edge_attr.T

    ne0, _ = params['node_enc'][0]
    ne1, _ = params['node_enc'][1]
    ee0, _ = params['edge_enc'][0]
    ee1, _ = params['edge_enc'][1]
    tne0 = ne0.T
    tne1p = _pad_rows(ne1.T, HP)
    tee0 = ee0.T
    tee1p = ee1.T

    h_t = _node_enc(x_t, tne0, tne1p)
    e_t = _edge_enc(ea_t, tee0, tee1p)
    e_list = [e_t]

    def rel_weights(lp):
        (rw1, rb1), (rw2, rb2), (rw3, rb3) = lp['rel']
        return (rw1.T,
                rb1[:, None], rw2.T, rb2[:, None], rw3.T, rb3[:, None])

    for lp in params['resin'][:-1]:
        (ow1, ob1), (ow2, ob2), (ow3, ob3) = lp['obj']
        g2 = _gather(h_t, src, dst)
        et2, e_t = _rel(g2, e_t, *rel_weights(lp))
        part = _scatter(et2, dst)

        tow1 = ow1.T
        towh = _pad_cols(tow1[:, 0:5], HP)
        towa = tow1[:, 5:9]
        tow3 = _pad_rows(ow3.T, HP)
        tob3 = _pad_rows(ob3[:, None], HP)
        h_t = _obj(h_t, part, towh, towa, ob1[:, None], ow2.T,
                   ob2[:, None], tow3, tob3)
        e_list.append(e_t)

    (ww1, wb1), (ww2, wb2), (ww3, wb3) = params['W']
    finw = (ww1.T,
            wb1[:, None], ww2.T, wb2[:, None], ww3.T, wb3[:, None])

    g2 = _gather(h_t, src, dst)
    w8 = _rel_final(g2, e_list, rel_weights(params['resin'][-1]), finw)
    return w8[0][:, None]
